# baseline (speedup 1.0000x reference)
import functools

import jax
import jax.numpy as jnp
from jax import lax
from jax.experimental import pallas as pl
from jax.experimental.pallas import tpu as pltpu
from jax.experimental.pallas import tpu_sc as plsc

B = 4096
CTX = 20
D = 128
V = 100000

NC = 2
NS = 16
NW = NC * NS
EPW = B // NW
CH = 4
IPC = CH * CTX
NCHUNK = EPW // CH


def _sc_gather(idx2d, table):
    mesh = plsc.VectorSubcoreMesh(core_axis_name="c", subcore_axis_name="s")

    @functools.partial(
        pl.kernel,
        mesh=mesh,
        out_type=jax.ShapeDtypeStruct((B * CTX, D), jnp.float32),
        scratch_types=[
            pltpu.VMEM((NCHUNK, IPC), jnp.int32),
            pltpu.VMEM((IPC, D), jnp.float32),
            pltpu.VMEM((IPC, D), jnp.float32),
            pltpu.SemaphoreType.DMA,
            pltpu.SemaphoreType.DMA,
            pltpu.SemaphoreType.DMA,
            pltpu.SemaphoreType.DMA,
        ],
    )
    def k(idx_hbm, table_hbm, out_hbm, idx_v, rows0, rows1, g0, g1, s0, s1):
        wid = lax.axis_index("s") * NC + lax.axis_index("c")
        pltpu.sync_copy(idx_hbm.at[pl.ds(wid * NCHUNK, NCHUNK)], idx_v)
        rows = (rows0, rows1)
        gsem = (g0, g1)
        ssem = (s0, s1)
        base_row = wid * EPW * CTX

        gh = {}
        sh = {}

        def start_gather(i):
            gh[i] = pltpu.async_copy(
                table_hbm.at[idx_v.at[i]], rows[i % 2], gsem[i % 2]
            )

        def start_store(i):
            sh[i] = pltpu.async_copy(
                rows[i % 2],
                out_hbm.at[pl.ds(base_row + i * IPC, IPC)],
                ssem[i % 2],
            )

        start_gather(0)
        for i in range(NCHUNK):
            gh[i].wait()
            if i + 1 < NCHUNK:
                if i >= 1:
                    sh[i - 1].wait()
                start_gather(i + 1)
            start_store(i)
        sh[NCHUNK - 2].wait()
        sh[NCHUNK - 1].wait()

    return k(idx2d, table)


def _mean_body(x_ref, o_ref):
    o_ref[...] = jnp.sum(x_ref[...], axis=1) * (1.0 / CTX)


def _matmul_body(m_ref, w_ref, b_ref, o_ref):
    acc = lax.dot_general(
        m_ref[...], w_ref[...], (((1,), (1,)), ((), ())),
        preferred_element_type=jnp.float32,
    )
    o_ref[...] = acc + b_ref[...]


def kernel(context_words, emb_table, lin_w, lin_b):
    idx2d = context_words.astype(jnp.int32).reshape(NW * NCHUNK, IPC)
    gathered = _sc_gather(idx2d, emb_table)

    TB = 512
    mean = pl.pallas_call(
        _mean_body,
        grid=(B // TB,),
        in_specs=[pl.BlockSpec((TB, CTX, D), lambda i: (i, 0, 0))],
        out_specs=pl.BlockSpec((TB, D), lambda i: (i, 0)),
        out_shape=jax.ShapeDtypeStruct((B, D), jnp.float32),
    )(gathered.reshape(B, CTX, D))

    TV = 1024
    scores = pl.pallas_call(
        _matmul_body,
        grid=(pl.cdiv(V, TV),),
        in_specs=[
            pl.BlockSpec((B, D), lambda i: (0, 0)),
            pl.BlockSpec((TV, D), lambda i: (i, 0)),
            pl.BlockSpec((1, TV), lambda i: (0, i)),
        ],
        out_specs=pl.BlockSpec((B, TV), lambda i: (0, i)),
        out_shape=jax.ShapeDtypeStruct((B, V), jnp.float32),
        compiler_params=pltpu.CompilerParams(
            dimension_semantics=("arbitrary",),
        ),
    )(mean, lin_w, lin_b.reshape(1, V))
    return scores


# speedup vs baseline: 1.0006x; 1.0006x over previous
import functools

import jax
import jax.numpy as jnp
from jax import lax
from jax.experimental import pallas as pl
from jax.experimental.pallas import tpu as pltpu
from jax.experimental.pallas import tpu_sc as plsc

B = 4096
CTX = 20
D = 128
V = 100000

NC = 2
NS = 16
NW = NC * NS
EPW = B // NW
CH = 4
IPC = CH * CTX
NCHUNK = EPW // CH


def _sc_gather(idx2d, table):
    mesh = plsc.VectorSubcoreMesh(core_axis_name="c", subcore_axis_name="s")

    @functools.partial(
        pl.kernel,
        mesh=mesh,
        out_type=jax.ShapeDtypeStruct((B * CTX, D), jnp.float32),
        scratch_types=[
            pltpu.VMEM((NCHUNK, IPC), jnp.int32),
            pltpu.VMEM((IPC, D), jnp.float32),
            pltpu.VMEM((IPC, D), jnp.float32),
            pltpu.SemaphoreType.DMA,
            pltpu.SemaphoreType.DMA,
            pltpu.SemaphoreType.DMA,
            pltpu.SemaphoreType.DMA,
        ],
    )
    def k(idx_hbm, table_hbm, out_hbm, idx_v, rows0, rows1, g0, g1, s0, s1):
        wid = lax.axis_index("s") * NC + lax.axis_index("c")
        pltpu.sync_copy(idx_hbm.at[pl.ds(wid * NCHUNK, NCHUNK)], idx_v)
        rows = (rows0, rows1)
        gsem = (g0, g1)
        ssem = (s0, s1)
        base_row = wid * EPW * CTX

        gh = {}
        sh = {}

        def start_gather(i):
            gh[i] = pltpu.async_copy(
                table_hbm.at[idx_v.at[i]], rows[i % 2], gsem[i % 2]
            )

        def start_store(i):
            sh[i] = pltpu.async_copy(
                rows[i % 2],
                out_hbm.at[pl.ds(base_row + i * IPC, IPC)],
                ssem[i % 2],
            )

        start_gather(0)
        for i in range(NCHUNK):
            gh[i].wait()
            if i + 1 < NCHUNK:
                if i >= 1:
                    sh[i - 1].wait()
                start_gather(i + 1)
            start_store(i)
        sh[NCHUNK - 2].wait()
        sh[NCHUNK - 1].wait()

    return k(idx2d, table)


def _mean_body(x_ref, o_ref):
    o_ref[...] = jnp.sum(x_ref[...], axis=1) * (1.0 / CTX)


def _matmul_body(m_ref, w_ref, b_ref, o_ref):
    acc = lax.dot_general(
        m_ref[...], w_ref[...], (((1,), (1,)), ((), ())),
        preferred_element_type=jnp.float32,
    )
    o_ref[...] = acc + b_ref[...]


def kernel(context_words, emb_table, lin_w, lin_b):
    idx2d = context_words.astype(jnp.int32).reshape(NW * NCHUNK, IPC)
    gathered = _sc_gather(idx2d, emb_table)

    TB = 512
    mean = pl.pallas_call(
        _mean_body,
        grid=(B // TB,),
        in_specs=[pl.BlockSpec((TB, CTX, D), lambda i: (i, 0, 0))],
        out_specs=pl.BlockSpec((TB, D), lambda i: (i, 0)),
        out_shape=jax.ShapeDtypeStruct((B, D), jnp.float32),
    )(gathered.reshape(B, CTX, D))

    TV = 1024
    scores = pl.pallas_call(
        _matmul_body,
        grid=(pl.cdiv(V, TV),),
        in_specs=[
            pl.BlockSpec((B, D), lambda i: (0, 0)),
            pl.BlockSpec((TV, D), lambda i: (i, 0)),
            pl.BlockSpec((1, TV), lambda i: (0, i)),
        ],
        out_specs=pl.BlockSpec((B, TV), lambda i: (0, i)),
        out_shape=jax.ShapeDtypeStruct((B, V), jnp.float32),
        compiler_params=pltpu.CompilerParams(
            dimension_semantics=("parallel",),
        ),
    )(mean, lin_w, lin_b.reshape(1, V))
    return scores
